# Initial kernel scaffold; baseline (speedup 1.0000x reference)
import jax
import jax.numpy as jnp
from jax.experimental import pallas as pl


def kernel(x, edge_index, W1, al1, ar1, W2, al2, ar2, W3, al3, ar3):
    raise NotImplementedError("write your pallas kernel here")


# speedup vs baseline: 16.7978x; 16.7978x over previous
import functools

import jax
import jax.numpy as jnp
from jax import lax
from jax.experimental import pallas as pl
from jax.experimental.pallas import tpu as pltpu
from jax.experimental.pallas import tpu_sc as plsc

N = 10000
NP = 10240
E = 320000
F = 128
FH = 64
PW = 80
NW = 32
EPW = E // NW
C = 400
SUB = 80
NSUB = C // SUB
NCH = EPW // C
GPC = C // 16
RPT = NP // 16
BN = 1280


def _expand(den, d):
    hp = den.shape[1]
    hi = lax.broadcasted_iota(jnp.int32, (hp, hp * d), 0)
    di = lax.broadcasted_iota(jnp.int32, (hp, hp * d), 1)
    m = jnp.where(di // d == hi, jnp.ones((), jnp.float32),
                  jnp.zeros((), jnp.float32))
    return jnp.dot(den, m, preferred_element_type=jnp.float32)


def _half_normalize(pa, pb, hp, relu):
    d = FH // hp
    s = pa + pb
    den = jnp.maximum(s[:, FH:FH + hp], 1e-9)
    y = s[:, :FH] / _expand(den, d)
    if relu:
        y = jnp.maximum(y, 0.0)
    return y


def _emit_pass_tables(y, wp_ref, alf_ref, arf_ref, t_ref, hp):
    d = FH // hp
    h = jnp.dot(y, wp_ref[...], preferred_element_type=jnp.float32)
    ri = lax.broadcasted_iota(jnp.int32, (FH, hp), 0)
    ci = lax.broadcasted_iota(jnp.int32, (FH, hp), 1)
    ml = jnp.where(ri // d == ci, alf_ref[...], jnp.zeros((), jnp.float32))
    mr = jnp.where(ri // d == ci, arf_ref[...], jnp.zeros((), jnp.float32))
    el = jnp.dot(h, ml, preferred_element_type=jnp.float32)
    er = jnp.dot(h, mr, preferred_element_type=jnp.float32)
    z = jnp.zeros((h.shape[0], 16 - hp), jnp.float32)
    t_ref[...] = jnp.concatenate([h, el, z], axis=1)
    return er


def _prep_body(refs, n_pass, hp, first, hp_prev):
    if first:
        ins = 1
        y = refs[0][...]
    else:
        ins = 2 * (F // FH)
        ya = _half_normalize(refs[0][...], refs[1][...], hp_prev, relu=True)
        yb = _half_normalize(refs[2][...], refs[3][...], hp_prev, relu=True)
        y = jnp.concatenate([ya, yb], axis=1)
    ers = []
    for p in range(n_pass):
        wp = refs[ins + 3 * p]
        alf = refs[ins + 3 * p + 1]
        arf = refs[ins + 3 * p + 2]
        t_ref = refs[ins + 3 * n_pass + p]
        ers.append(_emit_pass_tables(y, wp, alf, arf, t_ref, hp))
    er_ref = refs[ins + 4 * n_pass]
    bn = ers[0].shape[0]
    z = jnp.zeros((bn, 16 - hp * n_pass), jnp.float32)
    er_ref[...] = jnp.concatenate(ers + [z], axis=1)


def _row_spec(width):
    return pl.BlockSpec((BN, width), lambda i: (i, 0))


def _full_spec(shape):
    return pl.BlockSpec(shape, lambda i: tuple(0 for _ in shape))


def _tc_prep(prev, weights, n_pass, hp, first, hp_prev):
    def body(*refs):
        _prep_body(refs, n_pass, hp, first, hp_prev)

    in_specs = ([_row_spec(F)] if first else [_row_spec(PW)] * 4)
    in_specs += [_full_spec((F, FH)), _full_spec((FH, 1)),
                 _full_spec((FH, 1))] * n_pass
    out_specs = [_row_spec(PW)] * n_pass + [_row_spec(16)]
    out_shape = ([jax.ShapeDtypeStruct((NP, PW), jnp.float32)] * n_pass
                 + [jax.ShapeDtypeStruct((NP, 16), jnp.float32)])
    return pl.pallas_call(
        body,
        grid=(NP // BN,),
        in_specs=in_specs,
        out_specs=out_specs,
        out_shape=out_shape,
    )(*(list(prev) + list(weights)))


def _final_body(*refs):
    parts = []
    for p in range(4):
        s = refs[2 * p][...] + refs[2 * p + 1][...]
        den = jnp.maximum(s[:, FH:FH + 1], 1e-9)
        y = s[:, :FH] / _expand(den, FH)
        parts.append(jnp.sum(y, axis=0, keepdims=True))
    out_ref = refs[8]
    row = jnp.concatenate(parts, axis=1)

    @pl.when(pl.program_id(0) == 0)
    def _():
        out_ref[...] = jnp.zeros_like(out_ref)

    out_ref[...] += row * (1.0 / N)


def _tc_final(partials):
    return pl.pallas_call(
        _final_body,
        grid=(NP // BN,),
        in_specs=[_row_spec(PW)] * 8,
        out_specs=pl.BlockSpec((1, 256), lambda i: (0, 0)),
        out_shape=jax.ShapeDtypeStruct((1, 256), jnp.float32),
    )(*partials)


def _sc_body(t_hbm, er_hbm, src_hbm, dst_hbm, zeros_hbm, out_hbm,
             a_v, b_v, idx_s, idx_d, acc_sh, sem, *, hp, ho):
    d = FH // hp
    ci = lax.axis_index("c")
    si = lax.axis_index("s")
    wkr = ci * 16 + si

    pltpu.sync_copy(zeros_hbm.at[pl.ds(si * RPT, RPT)],
                    acc_sh.at[pl.ds(si * RPT, RPT)])
    plsc.subcore_barrier()

    base_row = wkr * (EPW // SUB)

    def chunk_body(c_i, carry):
        r0 = base_row + c_i * NSUB
        for j in range(NSUB):
            pltpu.sync_copy(src_hbm.at[r0 + j], idx_s[j])
            pltpu.sync_copy(dst_hbm.at[r0 + j], idx_d[j])
        descs = []
        for j in range(NSUB):
            descs.append(pltpu.async_copy(
                t_hbm.at[idx_s[j]], a_v.at[pl.ds(j * SUB, SUB)], sem))
            descs.append(pltpu.async_copy(
                er_hbm.at[idx_d[j]], b_v.at[pl.ds(j * SUB, SUB)], sem))
        for dd in descs:
            dd.wait()

        def group(g, cc):
            rows = g * 16 + lax.iota(jnp.int32, 16)
            ws = []
            for h in range(hp):
                el = plsc.load_gather(
                    a_v, [rows, jnp.full((16,), FH + h, jnp.int32)])
                er = plsc.load_gather(
                    b_v, [rows, jnp.full((16,), ho + h, jnp.int32)])
                s = el + er
                wv = jnp.exp(jnp.maximum(s, 0.2 * s))
                plsc.store_scatter(
                    a_v, [rows, jnp.full((16,), FH + h, jnp.int32)], wv)
                ws.append(wv)
            for col in range(FH):
                cvec = jnp.full((16,), col, jnp.int32)
                v = plsc.load_gather(a_v, [rows, cvec])
                plsc.store_scatter(a_v, [rows, cvec], v * ws[col // d])
            return cc

        lax.fori_loop(0, GPC, group, 0)

        for j in range(NSUB):
            pltpu.sync_copy(a_v.at[pl.ds(j * SUB, SUB)],
                            acc_sh.at[idx_d[j]], add=True)
        return carry

    lax.fori_loop(0, NCH, chunk_body, 0)
    plsc.subcore_barrier()
    pltpu.sync_copy(acc_sh.at[pl.ds(si * RPT, RPT)],
                    out_hbm.at[ci, pl.ds(si * RPT, RPT)])


@functools.lru_cache(maxsize=None)
def _make_sc_pass(hp, ho):
    mesh = plsc.VectorSubcoreMesh(core_axis_name="c", subcore_axis_name="s",
                                  num_cores=2, num_subcores=16)
    return pl.kernel(
        functools.partial(_sc_body, hp=hp, ho=ho),
        out_type=jax.ShapeDtypeStruct((2, NP, PW), jnp.float32),
        mesh=mesh,
        compiler_params=pltpu.CompilerParams(needs_layout_passes=False,
                                             use_tc_tiling_on_sc=False),
        scratch_types=[
            pltpu.VMEM((C, PW), jnp.float32),
            pltpu.VMEM((C, 16), jnp.float32),
            [pltpu.VMEM((SUB,), jnp.int32) for _ in range(NSUB)],
            [pltpu.VMEM((SUB,), jnp.int32) for _ in range(NSUB)],
            pltpu.VMEM_SHARED((NP, PW), jnp.float32),
            pltpu.SemaphoreType.DMA,
        ],
    )


@jax.jit
def kernel(x, edge_index, W1, al1, ar1, W2, al2, ar2, W3, al3, ar3):
    src2d = edge_index[0].reshape(E // SUB, SUB)
    dst2d = edge_index[1].reshape(E // SUB, SUB)
    zeros = jnp.zeros((NP, PW), jnp.float32)
    xp = jnp.pad(x, ((0, NP - N), (0, 0)))

    def half_weights(W, al, ar):
        out = []
        for p in range(2):
            out += [W[:, p * FH:(p + 1) * FH],
                    al[2 * p:2 * p + 2].reshape(FH, 1),
                    ar[2 * p:2 * p + 2].reshape(FH, 1)]
        return out

    def head_weights(W, al, ar):
        out = []
        for p in range(4):
            out += [W[:, p * FH:(p + 1) * FH],
                    al[p].reshape(FH, 1),
                    ar[p].reshape(FH, 1)]
        return out

    def run_layer(tables, er, hp):
        n_pass = len(tables)
        outs = []
        for p in range(n_pass):
            sc = _make_sc_pass(hp, p * hp)
            part = sc(tables[p], er, src2d, dst2d, zeros)
            outs += [part[0], part[1]]
        return outs

    *t1, er1 = _tc_prep([xp], half_weights(W1, al1, ar1), 2, 2, True, 0)
    p1 = run_layer(t1, er1, 2)
    *t2, er2 = _tc_prep(p1, half_weights(W2, al2, ar2), 2, 2, False, 2)
    p2 = run_layer(t2, er2, 2)
    *t3, er3 = _tc_prep(p2, head_weights(W3, al3, ar3), 4, 1, False, 2)
    p3 = run_layer(t3, er3, 1)
    return _tc_final(p3)


# speedup vs baseline: 19.9812x; 1.1895x over previous
import functools

import jax
import jax.numpy as jnp
from jax import lax
from jax.experimental import pallas as pl
from jax.experimental.pallas import tpu as pltpu
from jax.experimental.pallas import tpu_sc as plsc

N = 10000
NP = 10240
E = 320000
F = 128
FH = 64
PW = 72
NW = 32
EPW = E // NW
C = 400
SUB = 80
NSUB = C // SUB
NCH = EPW // C
GPC = C // 16
RPT = NP // 16
BN = 1280


def _expand(den, d):
    hp = den.shape[1]
    hi = lax.broadcasted_iota(jnp.int32, (hp, hp * d), 0)
    di = lax.broadcasted_iota(jnp.int32, (hp, hp * d), 1)
    m = jnp.where(di // d == hi, jnp.ones((), jnp.float32),
                  jnp.zeros((), jnp.float32))
    return jnp.dot(den, m, preferred_element_type=jnp.float32)


def _half_normalize(pa, pb, hp, relu):
    d = FH // hp
    s = pa + pb
    den = jnp.maximum(s[:, FH:FH + hp], 1e-9)
    y = s[:, :FH] / _expand(den, d)
    if relu:
        y = jnp.maximum(y, 0.0)
    return y


def _emit_pass_tables(y, wp_ref, alf_ref, arf_ref, t_ref, hp):
    d = FH // hp
    h = jnp.dot(y, wp_ref[...], preferred_element_type=jnp.float32)
    ri = lax.broadcasted_iota(jnp.int32, (FH, hp), 0)
    ci = lax.broadcasted_iota(jnp.int32, (FH, hp), 1)
    ml = jnp.where(ri // d == ci, alf_ref[...], jnp.zeros((), jnp.float32))
    mr = jnp.where(ri // d == ci, arf_ref[...], jnp.zeros((), jnp.float32))
    el = jnp.dot(h, ml, preferred_element_type=jnp.float32)
    er = jnp.dot(h, mr, preferred_element_type=jnp.float32)
    z = jnp.zeros((h.shape[0], PW - FH - hp), jnp.float32)
    t_ref[...] = jnp.concatenate([h, el, z], axis=1)
    return er


def _prep_body(refs, n_pass, hp, first, hp_prev):
    if first:
        ins = 1
        y = refs[0][...]
    else:
        ins = 2 * (F // FH)
        ya = _half_normalize(refs[0][...], refs[1][...], hp_prev, relu=True)
        yb = _half_normalize(refs[2][...], refs[3][...], hp_prev, relu=True)
        y = jnp.concatenate([ya, yb], axis=1)
    ers = []
    for p in range(n_pass):
        wp = refs[ins + 3 * p]
        alf = refs[ins + 3 * p + 1]
        arf = refs[ins + 3 * p + 2]
        t_ref = refs[ins + 3 * n_pass + p]
        ers.append(_emit_pass_tables(y, wp, alf, arf, t_ref, hp))
    er_ref = refs[ins + 4 * n_pass]
    bn = ers[0].shape[0]
    z = jnp.zeros((bn, 16 - hp * n_pass), jnp.float32)
    er_ref[...] = jnp.concatenate(ers + [z], axis=1)


def _row_spec(width):
    return pl.BlockSpec((BN, width), lambda i: (i, 0))


def _full_spec(shape):
    return pl.BlockSpec(shape, lambda i: tuple(0 for _ in shape))


def _tc_prep(prev, weights, n_pass, hp, first, hp_prev):
    def body(*refs):
        _prep_body(refs, n_pass, hp, first, hp_prev)

    in_specs = ([_row_spec(F)] if first else [_row_spec(PW)] * 4)
    in_specs += [_full_spec((F, FH)), _full_spec((FH, 1)),
                 _full_spec((FH, 1))] * n_pass
    out_specs = [_row_spec(PW)] * n_pass + [_row_spec(16)]
    out_shape = ([jax.ShapeDtypeStruct((NP, PW), jnp.float32)] * n_pass
                 + [jax.ShapeDtypeStruct((NP, 16), jnp.float32)])
    return pl.pallas_call(
        body,
        grid=(NP // BN,),
        in_specs=in_specs,
        out_specs=out_specs,
        out_shape=out_shape,
    )(*(list(prev) + list(weights)))


def _final_body(*refs):
    parts = []
    for p in range(4):
        s = refs[2 * p][...] + refs[2 * p + 1][...]
        den = jnp.maximum(s[:, FH:FH + 1], 1e-9)
        y = s[:, :FH] / _expand(den, FH)
        parts.append(jnp.sum(y, axis=0, keepdims=True))
    out_ref = refs[8]
    row = jnp.concatenate(parts, axis=1)

    @pl.when(pl.program_id(0) == 0)
    def _():
        out_ref[...] = jnp.zeros_like(out_ref)

    out_ref[...] += row * (1.0 / N)


def _tc_final(partials):
    return pl.pallas_call(
        _final_body,
        grid=(NP // BN,),
        in_specs=[_row_spec(PW)] * 8,
        out_specs=pl.BlockSpec((1, 256), lambda i: (0, 0)),
        out_shape=jax.ShapeDtypeStruct((1, 256), jnp.float32),
    )(*partials)


def _sc_body(t_hbm, er_hbm, src_hbm, dst_hbm, zeros_hbm, out_hbm,
             a_v, b_v, idx_s, idx_d, acc_sh, sem, *, hp, ho):
    d = FH // hp
    ci = lax.axis_index("c")
    si = lax.axis_index("s")
    wkr = ci * 16 + si

    pltpu.sync_copy(zeros_hbm.at[pl.ds(si * RPT, RPT)],
                    acc_sh.at[pl.ds(si * RPT, RPT)])
    plsc.subcore_barrier()

    base_row = wkr * (EPW // SUB)

    def chunk_body(c_i, carry):
        r0 = base_row + c_i * NSUB
        for j in range(NSUB):
            pltpu.sync_copy(src_hbm.at[r0 + j], idx_s[j])
            pltpu.sync_copy(dst_hbm.at[r0 + j], idx_d[j])
        descs = []
        for j in range(NSUB):
            descs.append(pltpu.async_copy(
                t_hbm.at[idx_s[j]], a_v.at[pl.ds(j * SUB, SUB)], sem))
            descs.append(pltpu.async_copy(
                er_hbm.at[idx_d[j]], b_v.at[pl.ds(j * SUB, SUB)], sem))
        for dd in descs:
            dd.wait()

        def group(g, cc):
            rows = g * 16 + lax.iota(jnp.int32, 16)
            ws = []
            for h in range(hp):
                el = plsc.load_gather(
                    a_v, [rows, jnp.full((16,), FH + h, jnp.int32)])
                er = plsc.load_gather(
                    b_v, [rows, jnp.full((16,), ho + h, jnp.int32)])
                s = el + er
                wv = jnp.exp(jnp.maximum(s, 0.2 * s))
                plsc.store_scatter(
                    a_v, [rows, jnp.full((16,), FH + h, jnp.int32)], wv)
                ws.append(wv)
            for col in range(FH):
                cvec = jnp.full((16,), col, jnp.int32)
                v = plsc.load_gather(a_v, [rows, cvec])
                plsc.store_scatter(a_v, [rows, cvec], v * ws[col // d])
            return cc

        lax.fori_loop(0, GPC, group, 0)

        for j in range(NSUB):
            pltpu.sync_copy(a_v.at[pl.ds(j * SUB, SUB)],
                            acc_sh.at[idx_d[j]], add=True)
        return carry

    lax.fori_loop(0, NCH, chunk_body, 0)
    plsc.subcore_barrier()
    pltpu.sync_copy(acc_sh.at[pl.ds(si * RPT, RPT)],
                    out_hbm.at[ci, pl.ds(si * RPT, RPT)])


@functools.lru_cache(maxsize=None)
def _make_sc_pass(hp, ho):
    mesh = plsc.VectorSubcoreMesh(core_axis_name="c", subcore_axis_name="s",
                                  num_cores=2, num_subcores=16)
    return pl.kernel(
        functools.partial(_sc_body, hp=hp, ho=ho),
        out_type=jax.ShapeDtypeStruct((2, NP, PW), jnp.float32),
        mesh=mesh,
        compiler_params=pltpu.CompilerParams(needs_layout_passes=False,
                                             use_tc_tiling_on_sc=False),
        scratch_types=[
            pltpu.VMEM((C, PW), jnp.float32),
            pltpu.VMEM((C, 16), jnp.float32),
            [pltpu.VMEM((SUB,), jnp.int32) for _ in range(NSUB)],
            [pltpu.VMEM((SUB,), jnp.int32) for _ in range(NSUB)],
            pltpu.VMEM_SHARED((NP, PW), jnp.float32),
            pltpu.SemaphoreType.DMA,
        ],
    )


@jax.jit
def kernel(x, edge_index, W1, al1, ar1, W2, al2, ar2, W3, al3, ar3):
    src2d = edge_index[0].reshape(E // SUB, SUB)
    dst2d = edge_index[1].reshape(E // SUB, SUB)
    zeros = jnp.zeros((NP, PW), jnp.float32)
    xp = jnp.pad(x, ((0, NP - N), (0, 0)))

    def half_weights(W, al, ar):
        out = []
        for p in range(2):
            out += [W[:, p * FH:(p + 1) * FH],
                    al[2 * p:2 * p + 2].reshape(FH, 1),
                    ar[2 * p:2 * p + 2].reshape(FH, 1)]
        return out

    def head_weights(W, al, ar):
        out = []
        for p in range(4):
            out += [W[:, p * FH:(p + 1) * FH],
                    al[p].reshape(FH, 1),
                    ar[p].reshape(FH, 1)]
        return out

    def run_layer(tables, er, hp):
        n_pass = len(tables)
        outs = []
        for p in range(n_pass):
            sc = _make_sc_pass(hp, p * hp)
            part = sc(tables[p], er, src2d, dst2d, zeros)
            outs += [part[0], part[1]]
        return outs

    *t1, er1 = _tc_prep([xp], half_weights(W1, al1, ar1), 2, 2, True, 0)
    p1 = run_layer(t1, er1, 2)
    *t2, er2 = _tc_prep(p1, half_weights(W2, al2, ar2), 2, 2, False, 2)
    p2 = run_layer(t2, er2, 2)
    *t3, er3 = _tc_prep(p2, head_weights(W3, al3, ar3), 4, 1, False, 2)
    p3 = run_layer(t3, er3, 1)
    return _tc_final(p3)


# speedup vs baseline: 20.2485x; 1.0134x over previous
import functools

import jax
import jax.numpy as jnp
from jax import lax
from jax.experimental import pallas as pl
from jax.experimental.pallas import tpu as pltpu
from jax.experimental.pallas import tpu_sc as plsc

N = 10000
NP = 10240
E = 320000
F = 128
FH = 64
PW = 72
NW = 32
EPW = E // NW
C = 400
SUB = 80
NSUB = C // SUB
NCH = EPW // C
GPC = C // 16
RPT = NP // 16
BN = 1280


def _expand(den, d):
    hp = den.shape[1]
    hi = lax.broadcasted_iota(jnp.int32, (hp, hp * d), 0)
    di = lax.broadcasted_iota(jnp.int32, (hp, hp * d), 1)
    m = jnp.where(di // d == hi, jnp.ones((), jnp.float32),
                  jnp.zeros((), jnp.float32))
    return jnp.dot(den, m, preferred_element_type=jnp.float32)


def _half_normalize(pa, pb, hp, relu):
    d = FH // hp
    s = pa + pb
    den = jnp.maximum(s[:, FH:FH + hp], 1e-9)
    y = s[:, :FH] / _expand(den, d)
    if relu:
        y = jnp.maximum(y, 0.0)
    return y


def _emit_pass_tables(y, wp_ref, alf_ref, arf_ref, t_ref, hp):
    d = FH // hp
    h = jnp.dot(y, wp_ref[...], preferred_element_type=jnp.float32)
    ri = lax.broadcasted_iota(jnp.int32, (FH, hp), 0)
    ci = lax.broadcasted_iota(jnp.int32, (FH, hp), 1)
    ml = jnp.where(ri // d == ci, alf_ref[...], jnp.zeros((), jnp.float32))
    mr = jnp.where(ri // d == ci, arf_ref[...], jnp.zeros((), jnp.float32))
    el = jnp.dot(h, ml, preferred_element_type=jnp.float32)
    er = jnp.dot(h, mr, preferred_element_type=jnp.float32)
    z = jnp.zeros((h.shape[0], PW - FH - hp), jnp.float32)
    t_ref[...] = jnp.concatenate([h, el, z], axis=1)
    return er


def _prep_body(refs, n_pass, hp, first, hp_prev):
    if first:
        ins = 1
        y = refs[0][...]
    else:
        ins = 2 * (F // FH)
        ya = _half_normalize(refs[0][...], refs[1][...], hp_prev, relu=True)
        yb = _half_normalize(refs[2][...], refs[3][...], hp_prev, relu=True)
        y = jnp.concatenate([ya, yb], axis=1)
    ers = []
    for p in range(n_pass):
        wp = refs[ins + 3 * p]
        alf = refs[ins + 3 * p + 1]
        arf = refs[ins + 3 * p + 2]
        t_ref = refs[ins + 3 * n_pass + p]
        ers.append(_emit_pass_tables(y, wp, alf, arf, t_ref, hp))
    er_ref = refs[ins + 4 * n_pass]
    bn = ers[0].shape[0]
    z = jnp.zeros((bn, 16 - hp * n_pass), jnp.float32)
    er_ref[...] = jnp.concatenate(ers + [z], axis=1)


def _row_spec(width):
    return pl.BlockSpec((BN, width), lambda i: (i, 0))


def _full_spec(shape):
    return pl.BlockSpec(shape, lambda i: tuple(0 for _ in shape))


def _tc_prep(prev, weights, n_pass, hp, first, hp_prev):
    def body(*refs):
        _prep_body(refs, n_pass, hp, first, hp_prev)

    in_specs = ([_row_spec(F)] if first else [_row_spec(PW)] * 4)
    in_specs += [_full_spec((F, FH)), _full_spec((FH, 1)),
                 _full_spec((FH, 1))] * n_pass
    out_specs = [_row_spec(PW)] * n_pass + [_row_spec(16)]
    out_shape = ([jax.ShapeDtypeStruct((NP, PW), jnp.float32)] * n_pass
                 + [jax.ShapeDtypeStruct((NP, 16), jnp.float32)])
    return pl.pallas_call(
        body,
        grid=(NP // BN,),
        in_specs=in_specs,
        out_specs=out_specs,
        out_shape=out_shape,
    )(*(list(prev) + list(weights)))


def _final_body(*refs):
    parts = []
    for p in range(4):
        s = refs[2 * p][...] + refs[2 * p + 1][...]
        den = jnp.maximum(s[:, FH:FH + 1], 1e-9)
        y = s[:, :FH] / _expand(den, FH)
        parts.append(jnp.sum(y, axis=0, keepdims=True))
    out_ref = refs[8]
    row = jnp.concatenate(parts, axis=1)

    @pl.when(pl.program_id(0) == 0)
    def _():
        out_ref[...] = jnp.zeros_like(out_ref)

    out_ref[...] += row * (1.0 / N)


def _tc_final(partials):
    return pl.pallas_call(
        _final_body,
        grid=(NP // BN,),
        in_specs=[_row_spec(PW)] * 8,
        out_specs=pl.BlockSpec((1, 256), lambda i: (0, 0)),
        out_shape=jax.ShapeDtypeStruct((1, 256), jnp.float32),
    )(*partials)


def _sc_body(t_hbm, er_hbm, src_hbm, dst_hbm, zeros_hbm, out_hbm,
             a_v, b_v, idx_s, idx_d, acc_sh, sem, *, hp, ho):
    d = FH // hp
    ci = lax.axis_index("c")
    si = lax.axis_index("s")
    wkr = ci * 16 + si

    pltpu.sync_copy(zeros_hbm.at[pl.ds(si * RPT, RPT)],
                    acc_sh.at[pl.ds(si * RPT, RPT)])
    plsc.subcore_barrier()

    base_row = wkr * (EPW // SUB)

    def chunk_body(c_i, carry):
        r0 = base_row + c_i * NSUB
        for j in range(NSUB):
            pltpu.sync_copy(src_hbm.at[r0 + j], idx_s[j])
            pltpu.sync_copy(dst_hbm.at[r0 + j], idx_d[j])
        descs = []
        for j in range(NSUB):
            descs.append(pltpu.async_copy(
                t_hbm.at[idx_s[j]], a_v.at[pl.ds(j * SUB, SUB)], sem))
            descs.append(pltpu.async_copy(
                er_hbm.at[idx_d[j]], b_v.at[pl.ds(j * SUB, SUB)], sem))
        for dd in descs:
            dd.wait()

        def group(g, cc):
            rows = g * 16 + lax.iota(jnp.int32, 16)
            ws = []
            for h in range(hp):
                el = plsc.load_gather(
                    a_v, [rows, jnp.full((16,), FH + h, jnp.int32)])
                er = plsc.load_gather(
                    b_v, [rows, jnp.full((16,), ho + h, jnp.int32)])
                s = el + er
                wv = jnp.exp(jnp.maximum(s, 0.2 * s))
                plsc.store_scatter(
                    a_v, [rows, jnp.full((16,), FH + h, jnp.int32)], wv)
                ws.append(wv)
            for col in range(FH):
                cvec = jnp.full((16,), col, jnp.int32)
                v = plsc.load_gather(a_v, [rows, cvec])
                plsc.store_scatter(a_v, [rows, cvec], v * ws[col // d])
            return cc

        lax.fori_loop(0, GPC, group, 0)

        sdescs = [pltpu.async_copy(a_v.at[pl.ds(j * SUB, SUB)],
                                   acc_sh.at[idx_d[j]], sem, add=True)
                  for j in range(NSUB)]
        for dd in sdescs:
            dd.wait()
        return carry

    lax.fori_loop(0, NCH, chunk_body, 0)
    plsc.subcore_barrier()
    pltpu.sync_copy(acc_sh.at[pl.ds(si * RPT, RPT)],
                    out_hbm.at[ci, pl.ds(si * RPT, RPT)])


@functools.lru_cache(maxsize=None)
def _make_sc_pass(hp, ho):
    mesh = plsc.VectorSubcoreMesh(core_axis_name="c", subcore_axis_name="s",
                                  num_cores=2, num_subcores=16)
    return pl.kernel(
        functools.partial(_sc_body, hp=hp, ho=ho),
        out_type=jax.ShapeDtypeStruct((2, NP, PW), jnp.float32),
        mesh=mesh,
        compiler_params=pltpu.CompilerParams(needs_layout_passes=False,
                                             use_tc_tiling_on_sc=False),
        scratch_types=[
            pltpu.VMEM((C, PW), jnp.float32),
            pltpu.VMEM((C, 16), jnp.float32),
            [pltpu.VMEM((SUB,), jnp.int32) for _ in range(NSUB)],
            [pltpu.VMEM((SUB,), jnp.int32) for _ in range(NSUB)],
            pltpu.VMEM_SHARED((NP, PW), jnp.float32),
            pltpu.SemaphoreType.DMA,
        ],
    )


@jax.jit
def kernel(x, edge_index, W1, al1, ar1, W2, al2, ar2, W3, al3, ar3):
    src2d = edge_index[0].reshape(E // SUB, SUB)
    dst2d = edge_index[1].reshape(E // SUB, SUB)
    zeros = jnp.zeros((NP, PW), jnp.float32)
    xp = jnp.pad(x, ((0, NP - N), (0, 0)))

    def half_weights(W, al, ar):
        out = []
        for p in range(2):
            out += [W[:, p * FH:(p + 1) * FH],
                    al[2 * p:2 * p + 2].reshape(FH, 1),
                    ar[2 * p:2 * p + 2].reshape(FH, 1)]
        return out

    def head_weights(W, al, ar):
        out = []
        for p in range(4):
            out += [W[:, p * FH:(p + 1) * FH],
                    al[p].reshape(FH, 1),
                    ar[p].reshape(FH, 1)]
        return out

    def run_layer(tables, er, hp):
        n_pass = len(tables)
        outs = []
        for p in range(n_pass):
            sc = _make_sc_pass(hp, p * hp)
            part = sc(tables[p], er, src2d, dst2d, zeros)
            outs += [part[0], part[1]]
        return outs

    *t1, er1 = _tc_prep([xp], half_weights(W1, al1, ar1), 2, 2, True, 0)
    p1 = run_layer(t1, er1, 2)
    *t2, er2 = _tc_prep(p1, half_weights(W2, al2, ar2), 2, 2, False, 2)
    p2 = run_layer(t2, er2, 2)
    *t3, er3 = _tc_prep(p2, head_weights(W3, al3, ar3), 4, 1, False, 2)
    p3 = run_layer(t3, er3, 1)
    return _tc_final(p3)
